# baseline (speedup 1.0000x reference)
import math

import jax
import jax.numpy as jnp
from jax.experimental import pallas as pl
from jax.experimental.pallas import tpu as pltpu


def _round_up(x: int, m: int) -> int:
    return ((x + m - 1) // m) * m


def _make_pipe_body(nb: int):
    def body(x1_hbm, x2_hbm, out_hbm, x1_buf, x2_buf, o_buf,
             in1_sem, in2_sem, out_sem):
        base = pl.program_id(0) * nb

        def dma_in(slot, step):
            b = base + step
            pltpu.make_async_copy(x1_hbm.at[b], x1_buf.at[slot],
                                  in1_sem.at[slot]).start()
            pltpu.make_async_copy(x2_hbm.at[b], x2_buf.at[slot],
                                  in2_sem.at[slot]).start()

        def wait_in(slot):
            pltpu.make_async_copy(x1_buf.at[slot], x1_buf.at[slot],
                                  in1_sem.at[slot]).wait()
            pltpu.make_async_copy(x2_buf.at[slot], x2_buf.at[slot],
                                  in2_sem.at[slot]).wait()

        def dma_out(slot, step):
            b = base + step
            pltpu.make_async_copy(o_buf.at[slot], out_hbm.at[b],
                                  out_sem.at[slot]).start()

        def wait_out(slot):
            pltpu.make_async_copy(o_buf.at[slot], o_buf.at[slot],
                                  out_sem.at[slot]).wait()

        dma_in(0, 0)

        def step_fn(step, carry):
            cur = jax.lax.rem(step, 2)
            nxt = jax.lax.rem(step + 1, 2)

            @pl.when(step + 1 < nb)
            def _():
                dma_in(nxt, step + 1)

            wait_in(cur)

            @pl.when(step >= 2)
            def _():
                wait_out(cur)

            a = x1_buf[cur].astype(jnp.bfloat16)
            bb = x2_buf[cur].astype(jnp.bfloat16)
            o_buf[cur, :, :] = jax.lax.dot_general(
                a,
                bb,
                dimension_numbers=(((1,), (1,)), ((), ())),
                preferred_element_type=jnp.float32,
            )
            dma_out(cur, step)
            return carry

        jax.lax.fori_loop(0, nb, step_fn, 0)
        wait_out(jax.lax.rem(nb - 2, 2))
        wait_out(jax.lax.rem(nb - 1, 2))

    return body


def _gram_pipelined(x1, x2, B, N_pad, M_pad, D_pad):
    nb = B // 2
    buf_bytes = 2 * (N_pad * D_pad + M_pad * D_pad + N_pad * M_pad) * 4
    return pl.pallas_call(
        _make_pipe_body(nb),
        out_shape=jax.ShapeDtypeStruct((B, N_pad, M_pad), jnp.float32),
        grid=(2,),
        in_specs=[
            pl.BlockSpec(memory_space=pl.ANY),
            pl.BlockSpec(memory_space=pl.ANY),
        ],
        out_specs=pl.BlockSpec(memory_space=pl.ANY),
        scratch_shapes=[
            pltpu.VMEM((2, N_pad, D_pad), jnp.float32),
            pltpu.VMEM((2, M_pad, D_pad), jnp.float32),
            pltpu.VMEM((2, N_pad, M_pad), jnp.float32),
            pltpu.SemaphoreType.DMA((2,)),
            pltpu.SemaphoreType.DMA((2,)),
            pltpu.SemaphoreType.DMA((2,)),
        ],
        compiler_params=pltpu.CompilerParams(
            dimension_semantics=("parallel",),
            vmem_limit_bytes=int(min(60 * 1024 * 1024, buf_bytes + (1 << 20))),
        ),
        cost_estimate=pl.CostEstimate(
            flops=2 * B * N_pad * M_pad * D_pad,
            transcendentals=0,
            bytes_accessed=4 * B * ((N_pad + M_pad) * D_pad + N_pad * M_pad),
        ),
    )(x1, x2)


def _gram_block_body(x1_ref, x2_ref, out_ref):
    a = x1_ref[...].astype(jnp.bfloat16)
    b = x2_ref[...].astype(jnp.bfloat16)
    out_ref[...] = jax.lax.dot_general(
        a,
        b,
        dimension_numbers=(((2,), (2,)), ((0,), (0,))),
        preferred_element_type=jnp.float32,
    )


def _gram_blocked(x1, x2, B, N_pad, M_pad, D_pad):
    per_batch_bytes = (N_pad * D_pad + M_pad * D_pad + N_pad * M_pad) * 4
    bt = max(1, min(B, (15 * 1024 * 1024) // max(per_batch_bytes, 1)))
    if B > 1:
        bt = min(bt, max(1, B // 2))
    while B % bt:
        bt -= 1
    steps = B // bt
    block_bytes = bt * per_batch_bytes
    return pl.pallas_call(
        _gram_block_body,
        out_shape=jax.ShapeDtypeStruct((B, N_pad, M_pad), jnp.float32),
        grid=(steps,),
        in_specs=[
            pl.BlockSpec((bt, N_pad, D_pad), lambda i: (i, 0, 0)),
            pl.BlockSpec((bt, M_pad, D_pad), lambda i: (i, 0, 0)),
        ],
        out_specs=pl.BlockSpec((bt, N_pad, M_pad), lambda i: (i, 0, 0)),
        compiler_params=pltpu.CompilerParams(
            dimension_semantics=("parallel",),
            vmem_limit_bytes=int(
                min(60 * 1024 * 1024, max(16 * 1024 * 1024, 3 * block_bytes))
            ),
        ),
        cost_estimate=pl.CostEstimate(
            flops=2 * B * N_pad * M_pad * D_pad,
            transcendentals=0,
            bytes_accessed=4 * B * ((N_pad + M_pad) * D_pad + N_pad * M_pad),
        ),
    )(x1, x2)


def kernel(X1: jax.Array, X2: jax.Array) -> jax.Array:
    if X1.shape[-1] != X2.shape[-1]:
        raise ValueError(
            f"Input vectors must have the same feature dimension. "
            f"Got X1 dim {X1.shape[-1]} and X2 dim {X2.shape[-1]}"
        )

    N, D = X1.shape[-2], X1.shape[-1]
    M = X2.shape[-2]
    batch_shape = jnp.broadcast_shapes(X1.shape[:-2], X2.shape[:-2])
    B = math.prod(batch_shape) if batch_shape else 1

    x1 = jnp.broadcast_to(X1.astype(jnp.float32), (*batch_shape, N, D))
    x2 = jnp.broadcast_to(X2.astype(jnp.float32), (*batch_shape, M, D))
    x1 = x1.reshape(B, N, D)
    x2 = x2.reshape(B, M, D)

    N_pad = _round_up(N, 8)
    M_pad = _round_up(M, 128)
    D_pad = _round_up(D, 128)

    def _pad(x, rows, rows_pad):
        pads = ((0, 0), (0, rows_pad - rows), (0, D_pad - D))
        return jnp.pad(x, pads) if any(p[1] for p in pads) else x

    x1p = _pad(x1, N, N_pad)
    x2p = _pad(x2, M, M_pad)

    pipe_buf_bytes = 2 * (N_pad * D_pad + M_pad * D_pad + N_pad * M_pad) * 4
    if B % 2 == 0 and B >= 4 and pipe_buf_bytes <= 56 * 1024 * 1024:
        out = _gram_pipelined(x1p, x2p, B, N_pad, M_pad, D_pad)
    else:
        out = _gram_blocked(x1p, x2p, B, N_pad, M_pad, D_pad)

    out = out[:, :N, :M]
    return out.reshape(*batch_shape, N, M)


# speedup vs baseline: 1.1722x; 1.1722x over previous
import math

import jax
import jax.numpy as jnp
from jax.experimental import pallas as pl
from jax.experimental.pallas import tpu as pltpu


def _round_up(x: int, m: int) -> int:
    return ((x + m - 1) // m) * m


def _make_pipe_body(nb: int):
    def body(x1_hbm, x2_hbm, out_hbm, x1_buf, x2_buf, o_buf,
             in1_sem, in2_sem, out_sem):
        base = pl.program_id(0) * nb

        def dma_in(slot, step):
            b = base + step
            pltpu.make_async_copy(x1_hbm.at[b], x1_buf.at[slot],
                                  in1_sem.at[slot]).start()
            pltpu.make_async_copy(x2_hbm.at[b], x2_buf.at[slot],
                                  in2_sem.at[slot]).start()

        def wait_in(slot):
            pltpu.make_async_copy(x1_buf.at[slot], x1_buf.at[slot],
                                  in1_sem.at[slot]).wait()
            pltpu.make_async_copy(x2_buf.at[slot], x2_buf.at[slot],
                                  in2_sem.at[slot]).wait()

        def dma_out(slot, step):
            b = base + step
            pltpu.make_async_copy(o_buf.at[slot], out_hbm.at[b],
                                  out_sem.at[slot]).start()

        def wait_out(slot):
            pltpu.make_async_copy(o_buf.at[slot], o_buf.at[slot],
                                  out_sem.at[slot]).wait()

        dma_in(0, 0)

        def step_fn(step, carry):
            cur = jax.lax.rem(step, 2)
            nxt = jax.lax.rem(step + 1, 2)

            @pl.when(step + 1 < nb)
            def _():
                dma_in(nxt, step + 1)

            wait_in(cur)

            @pl.when(step >= 2)
            def _():
                wait_out(cur)

            a = x1_buf[cur].astype(jnp.bfloat16)
            bb = x2_buf[cur].astype(jnp.bfloat16)
            o_buf[cur, :, :] = jax.lax.dot_general(
                a,
                bb,
                dimension_numbers=(((1,), (1,)), ((), ())),
                preferred_element_type=jnp.float32,
            )
            dma_out(cur, step)
            return carry

        jax.lax.fori_loop(0, nb, step_fn, 0)
        wait_out(jax.lax.rem(nb - 2, 2))
        wait_out(jax.lax.rem(nb - 1, 2))

    return body


def _gram_pipelined(x1, x2, B, N_pad, M_pad, D_pad):
    nb = B // 2
    buf_bytes = 2 * (N_pad * D_pad + M_pad * D_pad + N_pad * M_pad) * 4
    return pl.pallas_call(
        _make_pipe_body(nb),
        out_shape=jax.ShapeDtypeStruct((B, N_pad, M_pad), jnp.float32),
        grid=(2,),
        in_specs=[
            pl.BlockSpec(memory_space=pl.ANY),
            pl.BlockSpec(memory_space=pl.ANY),
        ],
        out_specs=pl.BlockSpec(memory_space=pl.ANY),
        scratch_shapes=[
            pltpu.VMEM((2, N_pad, D_pad), jnp.float32),
            pltpu.VMEM((2, M_pad, D_pad), jnp.float32),
            pltpu.VMEM((2, N_pad, M_pad), jnp.float32),
            pltpu.SemaphoreType.DMA((2,)),
            pltpu.SemaphoreType.DMA((2,)),
            pltpu.SemaphoreType.DMA((2,)),
        ],
        compiler_params=pltpu.CompilerParams(
            dimension_semantics=("parallel",),
            vmem_limit_bytes=int(min(60 * 1024 * 1024, buf_bytes + (1 << 20))),
        ),
        cost_estimate=pl.CostEstimate(
            flops=2 * B * N_pad * M_pad * D_pad,
            transcendentals=0,
            bytes_accessed=4 * B * ((N_pad + M_pad) * D_pad + N_pad * M_pad),
        ),
    )(x1, x2)


def _gram_ksplit_body(x1_ref, x2_ref, out_ref):
    k = pl.program_id(1)

    def _partial():
        return jax.lax.dot_general(
            x1_ref[...].astype(jnp.bfloat16),
            x2_ref[...].astype(jnp.bfloat16),
            dimension_numbers=(((2,), (2,)), ((0,), (0,))),
            preferred_element_type=jnp.float32,
        )

    @pl.when(k == 0)
    def _():
        out_ref[...] = _partial()

    @pl.when(k != 0)
    def _():
        out_ref[...] += _partial()


def _gram_ksplit(x1, x2, B, N_pad, M_pad, D_pad, tk):
    bt = B // 2
    return pl.pallas_call(
        _gram_ksplit_body,
        out_shape=jax.ShapeDtypeStruct((B, N_pad, M_pad), jnp.float32),
        grid=(2, D_pad // tk),
        in_specs=[
            pl.BlockSpec((bt, N_pad, tk), lambda i, k: (i, 0, k)),
            pl.BlockSpec((bt, M_pad, tk), lambda i, k: (i, 0, k)),
        ],
        out_specs=pl.BlockSpec((bt, N_pad, M_pad), lambda i, k: (i, 0, 0)),
        compiler_params=pltpu.CompilerParams(
            dimension_semantics=("parallel", "arbitrary"),
            vmem_limit_bytes=60 * 1024 * 1024,
        ),
        cost_estimate=pl.CostEstimate(
            flops=2 * B * N_pad * M_pad * D_pad,
            transcendentals=0,
            bytes_accessed=4 * B * ((N_pad + M_pad) * D_pad + N_pad * M_pad),
        ),
    )(x1, x2)


def _gram_block_body(x1_ref, x2_ref, out_ref):
    a = x1_ref[...].astype(jnp.bfloat16)
    b = x2_ref[...].astype(jnp.bfloat16)
    out_ref[...] = jax.lax.dot_general(
        a,
        b,
        dimension_numbers=(((2,), (2,)), ((0,), (0,))),
        preferred_element_type=jnp.float32,
    )


def _gram_blocked(x1, x2, B, N_pad, M_pad, D_pad):
    per_batch_bytes = (N_pad * D_pad + M_pad * D_pad + N_pad * M_pad) * 4
    bt = max(1, min(B, (15 * 1024 * 1024) // max(per_batch_bytes, 1)))
    if B > 1:
        bt = min(bt, max(1, B // 2))
    while B % bt:
        bt -= 1
    steps = B // bt
    block_bytes = bt * per_batch_bytes
    return pl.pallas_call(
        _gram_block_body,
        out_shape=jax.ShapeDtypeStruct((B, N_pad, M_pad), jnp.float32),
        grid=(steps,),
        in_specs=[
            pl.BlockSpec((bt, N_pad, D_pad), lambda i: (i, 0, 0)),
            pl.BlockSpec((bt, M_pad, D_pad), lambda i: (i, 0, 0)),
        ],
        out_specs=pl.BlockSpec((bt, N_pad, M_pad), lambda i: (i, 0, 0)),
        compiler_params=pltpu.CompilerParams(
            dimension_semantics=("parallel",),
            vmem_limit_bytes=int(
                min(60 * 1024 * 1024, max(16 * 1024 * 1024, 3 * block_bytes))
            ),
        ),
        cost_estimate=pl.CostEstimate(
            flops=2 * B * N_pad * M_pad * D_pad,
            transcendentals=0,
            bytes_accessed=4 * B * ((N_pad + M_pad) * D_pad + N_pad * M_pad),
        ),
    )(x1, x2)


def kernel(X1: jax.Array, X2: jax.Array) -> jax.Array:
    if X1.shape[-1] != X2.shape[-1]:
        raise ValueError(
            f"Input vectors must have the same feature dimension. "
            f"Got X1 dim {X1.shape[-1]} and X2 dim {X2.shape[-1]}"
        )

    N, D = X1.shape[-2], X1.shape[-1]
    M = X2.shape[-2]
    batch_shape = jnp.broadcast_shapes(X1.shape[:-2], X2.shape[:-2])
    B = math.prod(batch_shape) if batch_shape else 1

    x1 = jnp.broadcast_to(X1.astype(jnp.float32), (*batch_shape, N, D))
    x2 = jnp.broadcast_to(X2.astype(jnp.float32), (*batch_shape, M, D))
    x1 = x1.reshape(B, N, D)
    x2 = x2.reshape(B, M, D)

    N_pad = _round_up(N, 8)
    M_pad = _round_up(M, 128)
    D_pad = _round_up(D, 128)

    def _pad(x, rows, rows_pad):
        pads = ((0, 0), (0, rows_pad - rows), (0, D_pad - D))
        return jnp.pad(x, pads) if any(p[1] for p in pads) else x

    x1p = _pad(x1, N, N_pad)
    x2p = _pad(x2, M, M_pad)

    if B % 2 == 0 and B >= 4 and D_pad % 256 == 0 and (
        (B // 2) * (N_pad * 256 + M_pad * 256 + N_pad * M_pad) * 8 <= 56 * 1024 * 1024
    ):
        out = _gram_ksplit(x1p, x2p, B, N_pad, M_pad, D_pad, 256)
    else:
        out = _gram_blocked(x1p, x2p, B, N_pad, M_pad, D_pad)

    out = out[:, :N, :M]
    return out.reshape(*batch_shape, N, M)
